# baseline (speedup 1.0000x reference)
import functools

import jax
import jax.numpy as jnp
from jax import lax
from jax.experimental import pallas as pl
from jax.experimental.pallas import tpu as pltpu
from jax.experimental.pallas import tpu_sc as plsc

N = 4096
S = 7781
IMG = 3 * 64 * 64
KEPT = N - N // 8
NC, NS, L = 2, 16, 16
NW = NC * NS
OUT_PER_W = KEPT // NW
VREGS_PER_W = OUT_PER_W // L

_MASK_BR = 128


def _mask_body(spec_ref, mask_ref):
    x = spec_ref[...]
    nz = jnp.any(x != 0.0, axis=1)
    mask_ref[0, 0, :] = nz.astype(jnp.int32)


def _compute_mask(spectrum):
    nb = N // _MASK_BR
    mask3 = pl.pallas_call(
        _mask_body,
        grid=(nb,),
        in_specs=[pl.BlockSpec((_MASK_BR, S), lambda i: (i, 0))],
        out_specs=pl.BlockSpec((1, 1, _MASK_BR), lambda i: (i, 0, 0)),
        out_shape=jax.ShapeDtypeStruct((nb, 1, _MASK_BR), jnp.int32),
    )(spectrum)
    return mask3.reshape(N)


_mesh = plsc.VectorSubcoreMesh(core_axis_name="c", subcore_axis_name="s")


@functools.partial(
    pl.kernel,
    out_type=(
        jax.ShapeDtypeStruct((KEPT, S), jnp.float32),
        jax.ShapeDtypeStruct((KEPT, IMG), jnp.float32),
        jax.ShapeDtypeStruct((KEPT,), jnp.int32),
        jax.ShapeDtypeStruct((KEPT,), jnp.float32),
    ),
    mesh=_mesh,
    scratch_types=[
        pltpu.VMEM((N,), jnp.int32),
        pltpu.VMEM((KEPT,), jnp.int32),
        pltpu.VMEM((N,), jnp.int32),
        pltpu.VMEM((N,), jnp.float32),
        pltpu.VMEM((OUT_PER_W,), jnp.int32),
        pltpu.VMEM((OUT_PER_W,), jnp.float32),
        pltpu.SemaphoreType.DMA,
        pltpu.SemaphoreType.DMA,
    ],
    compiler_params=pltpu.CompilerParams(needs_layout_passes=False),
)
def _sc_compact(mask_hbm, spec_hbm, img_hbm, tid_hbm, rs_hbm,
                spec_out, img_out, tid_out, rs_out,
                mask_v, idx_v, tid_v, rs_v, tid_ov, rs_ov, sem_s, sem_i):
    wid = lax.axis_index("s") * NC + lax.axis_index("c")
    obase = wid * OUT_PER_W

    pltpu.sync_copy(mask_hbm, mask_v)
    pltpu.sync_copy(tid_hbm, tid_v)
    pltpu.sync_copy(rs_hbm, rs_v)

    zeros16 = jnp.zeros((L,), jnp.int32)

    def _zero(i, carry):
        idx_v[pl.ds(i * L, L)] = zeros16
        return carry

    lax.fori_loop(0, KEPT // L, _zero, 0)

    iota16 = lax.iota(jnp.int32, L)

    def _scan(c, carry):
        m = mask_v[pl.ds(c * L, L)]
        s = plsc.cumsum(m)
        pos = carry + s - m
        vals = c * L + iota16
        plsc.store_scatter(idx_v, [pos], vals, mask=m != 0)
        return carry + jnp.sum(m)

    lax.fori_loop(0, N // L, _scan, jnp.int32(0))

    for v in range(VREGS_PER_W):
        vec = idx_v[pl.ds(obase + v * L, L)]
        for k in range(L):
            r = vec[k]
            o = obase + v * L + k
            pltpu.async_copy(spec_hbm.at[pl.ds(r, 1)],
                             spec_out.at[pl.ds(o, 1)], sem_s)
            pltpu.async_copy(img_hbm.at[pl.ds(r, 1)],
                             img_out.at[pl.ds(o, 1)], sem_i)

    for v in range(VREGS_PER_W):
        ids = idx_v[pl.ds(obase + v * L, L)]
        tid_ov[pl.ds(v * L, L)] = plsc.load_gather(tid_v, [ids])
        rs_ov[pl.ds(v * L, L)] = plsc.load_gather(rs_v, [ids])
    pltpu.sync_copy(tid_ov, tid_out.at[pl.ds(obase, OUT_PER_W)])
    pltpu.sync_copy(rs_ov, rs_out.at[pl.ds(obase, OUT_PER_W)])

    def _drain(j, carry):
        pltpu.make_async_copy(spec_hbm.at[pl.ds(0, 1)],
                              spec_out.at[pl.ds(0, 1)], sem_s).wait()
        pltpu.make_async_copy(img_hbm.at[pl.ds(0, 1)],
                              img_out.at[pl.ds(0, 1)], sem_i).wait()
        return carry

    lax.fori_loop(0, OUT_PER_W, _drain, 0)


def kernel(spectrum, image, targetid, redshift):
    mask = _compute_mask(spectrum)
    img2 = image.reshape(N, IMG)
    spec_o, img_o, tid_o, rs_o = _sc_compact(mask, spectrum, img2,
                                             targetid, redshift)
    return spec_o, img_o.reshape(KEPT, 3, 64, 64), tid_o, rs_o


# speedup vs baseline: 2.2618x; 2.2618x over previous
import functools

import jax
import jax.numpy as jnp
from jax import lax
from jax.experimental import pallas as pl
from jax.experimental.pallas import tpu as pltpu
from jax.experimental.pallas import tpu_sc as plsc

N = 4096
S = 7781
IMG = 3 * 64 * 64
KEPT = N - N // 8
NC, NS, L = 2, 16, 16
NW = NC * NS
OUT_PER_W = KEPT // NW
CHUNK = 8
CHUNKS = OUT_PER_W // CHUNK

_MASK_BR = 128


def _mask_body(spec_ref, mask_ref):
    x = spec_ref[...]
    nz = jnp.any(x != 0.0, axis=1)
    mask_ref[0, 0, :] = nz.astype(jnp.int32)


def _compute_mask(spectrum):
    nb = N // _MASK_BR
    mask3 = pl.pallas_call(
        _mask_body,
        grid=(nb,),
        in_specs=[pl.BlockSpec((_MASK_BR, S), lambda i: (i, 0))],
        out_specs=pl.BlockSpec((1, 1, _MASK_BR), lambda i: (i, 0, 0)),
        out_shape=jax.ShapeDtypeStruct((nb, 1, _MASK_BR), jnp.int32),
    )(spectrum)
    return mask3.reshape(N)


_mesh = plsc.VectorSubcoreMesh(core_axis_name="c", subcore_axis_name="s")


@functools.partial(
    pl.kernel,
    out_type=(
        jax.ShapeDtypeStruct((KEPT,), jnp.int32),
        jax.ShapeDtypeStruct((KEPT, IMG), jnp.float32),
        jax.ShapeDtypeStruct((KEPT,), jnp.int32),
        jax.ShapeDtypeStruct((KEPT,), jnp.float32),
    ),
    mesh=_mesh,
    scratch_types=[
        pltpu.VMEM((N,), jnp.int32),
        pltpu.VMEM((KEPT,), jnp.int32),
        pltpu.VMEM((CHUNK, IMG), jnp.float32),
        pltpu.VMEM((N,), jnp.int32),
        pltpu.VMEM((N,), jnp.float32),
        pltpu.VMEM((OUT_PER_W,), jnp.int32),
        pltpu.VMEM((OUT_PER_W,), jnp.float32),
        pltpu.SemaphoreType.DMA,
    ],
    compiler_params=pltpu.CompilerParams(needs_layout_passes=False),
)
def _sc_compact(mask_hbm, img_hbm, tid_hbm, rs_hbm,
                idx_out, img_out, tid_out, rs_out,
                mask_v, idx_v, img_buf, tid_v, rs_v,
                tid_ov, rs_ov, sem_g):
    wid = lax.axis_index("s") * NC + lax.axis_index("c")
    obase = wid * OUT_PER_W

    pltpu.sync_copy(mask_hbm, mask_v)
    pltpu.sync_copy(tid_hbm, tid_v)
    pltpu.sync_copy(rs_hbm, rs_v)

    zeros16 = jnp.zeros((L,), jnp.int32)

    def _zero(i, carry):
        idx_v[pl.ds(i * L, L)] = zeros16
        return carry

    lax.fori_loop(0, KEPT // L, _zero, 0)

    iota16 = lax.iota(jnp.int32, L)

    def _scan(c, carry):
        m = mask_v[pl.ds(c * L, L)]
        s = plsc.cumsum(m)
        pos = carry + s - m
        vals = c * L + iota16
        plsc.store_scatter(idx_v, [pos], vals, mask=m != 0)
        return carry + jnp.sum(m)

    lax.fori_loop(0, N // L, _scan, jnp.int32(0))

    pltpu.sync_copy(idx_v.at[pl.ds(obase, OUT_PER_W)],
                    idx_out.at[pl.ds(obase, OUT_PER_W)])

    for c in range(CHUNKS):
        pltpu.async_copy(
            img_hbm.at[idx_v.at[pl.ds(obase + c * CHUNK, CHUNK)]],
            img_buf, sem_g).wait()
        pltpu.sync_copy(img_buf,
                        img_out.at[pl.ds(obase + c * CHUNK, CHUNK)])

    for v in range(OUT_PER_W // L):
        ids = idx_v[pl.ds(obase + v * L, L)]
        tid_ov[pl.ds(v * L, L)] = plsc.load_gather(tid_v, [ids])
        rs_ov[pl.ds(v * L, L)] = plsc.load_gather(rs_v, [ids])
    pltpu.sync_copy(tid_ov, tid_out.at[pl.ds(obase, OUT_PER_W)])
    pltpu.sync_copy(rs_ov, rs_out.at[pl.ds(obase, OUT_PER_W)])


def _spec_dma_body(idx_ref, spec_ref, out_ref, sem):
    def _issue(j, carry):
        r = idx_ref[j]
        pltpu.make_async_copy(spec_ref.at[pl.ds(r, 1)],
                              out_ref.at[pl.ds(j, 1)], sem).start()
        return carry

    lax.fori_loop(0, KEPT, _issue, 0)

    def _drain(j, carry):
        pltpu.make_async_copy(spec_ref.at[pl.ds(0, 1)],
                              out_ref.at[pl.ds(0, 1)], sem).wait()
        return carry

    lax.fori_loop(0, KEPT, _drain, 0)


def _gather_spectrum(idx, spectrum):
    return pl.pallas_call(
        _spec_dma_body,
        in_specs=[pl.BlockSpec(memory_space=pltpu.SMEM),
                  pl.BlockSpec(memory_space=pl.ANY)],
        out_specs=pl.BlockSpec(memory_space=pl.ANY),
        out_shape=jax.ShapeDtypeStruct((KEPT, S), jnp.float32),
        scratch_shapes=[pltpu.SemaphoreType.DMA],
    )(idx, spectrum)


def kernel(spectrum, image, targetid, redshift):
    mask = _compute_mask(spectrum)
    img2 = image.reshape(N, IMG)
    idx, img_o, tid_o, rs_o = _sc_compact(mask, img2, targetid, redshift)
    spec_o = _gather_spectrum(idx, spectrum)
    return spec_o, img_o.reshape(KEPT, 3, 64, 64), tid_o, rs_o


# speedup vs baseline: 6.8868x; 3.0448x over previous
import functools

import jax
import jax.numpy as jnp
from jax import lax
from jax.experimental import pallas as pl
from jax.experimental.pallas import tpu as pltpu
from jax.experimental.pallas import tpu_sc as plsc

N = 4096
S = 7781
SP = 7808
IMG = 3 * 64 * 64
IMGH = IMG // 2
KEPT = N - N // 8
NC, NS, L = 2, 16, 16
NW = NC * NS
OUT_PER_W = KEPT // NW
CHUNK = 8
CHUNKS = OUT_PER_W // CHUNK
HCHUNKS = 2 * CHUNKS

_MASK_BR = 128


def _mask_pad_body(spec_ref, mask_ref, pad_ref):
    x = spec_ref[...]
    nz = jnp.any(x != 0.0, axis=1)
    mask_ref[0, 0, :] = nz.astype(jnp.int32)
    pad_ref[:, :S] = x
    pad_ref[:, S:] = jnp.zeros((_MASK_BR, SP - S), jnp.float32)


def _compute_mask_pad(spectrum):
    nb = N // _MASK_BR
    mask3, spec_pad = pl.pallas_call(
        _mask_pad_body,
        grid=(nb,),
        in_specs=[pl.BlockSpec((_MASK_BR, S), lambda i: (i, 0))],
        out_specs=[pl.BlockSpec((1, 1, _MASK_BR), lambda i: (i, 0, 0)),
                   pl.BlockSpec((_MASK_BR, SP), lambda i: (i, 0))],
        out_shape=[jax.ShapeDtypeStruct((nb, 1, _MASK_BR), jnp.int32),
                   jax.ShapeDtypeStruct((N, SP), jnp.float32)],
    )(spectrum)
    return mask3.reshape(N), spec_pad


_mesh = plsc.VectorSubcoreMesh(core_axis_name="c", subcore_axis_name="s")
_sc_params = pltpu.CompilerParams(needs_layout_passes=False)


@functools.partial(
    pl.kernel,
    out_type=(
        jax.ShapeDtypeStruct((KEPT,), jnp.int32),
        jax.ShapeDtypeStruct((2 * KEPT,), jnp.int32),
        jax.ShapeDtypeStruct((KEPT,), jnp.int32),
        jax.ShapeDtypeStruct((KEPT,), jnp.float32),
    ),
    mesh=_mesh,
    scratch_types=[
        pltpu.VMEM((N,), jnp.int32),
        pltpu.VMEM((KEPT,), jnp.int32),
        pltpu.VMEM((2 * KEPT,), jnp.int32),
        pltpu.VMEM((N,), jnp.int32),
        pltpu.VMEM((N,), jnp.float32),
        pltpu.VMEM((OUT_PER_W,), jnp.int32),
        pltpu.VMEM((OUT_PER_W,), jnp.float32),
    ],
    compiler_params=_sc_params,
)
def _sc_scan(mask_hbm, tid_hbm, rs_hbm,
             idx_out, idx2_out, tid_out, rs_out,
             mask_v, idx_v, idx2_v, tid_v, rs_v, tid_ov, rs_ov):
    wid = lax.axis_index("s") * NC + lax.axis_index("c")
    obase = wid * OUT_PER_W

    pltpu.sync_copy(mask_hbm, mask_v)
    pltpu.sync_copy(tid_hbm, tid_v)
    pltpu.sync_copy(rs_hbm, rs_v)

    zeros16 = jnp.zeros((L,), jnp.int32)

    def _zero(i, carry):
        idx_v[pl.ds(i * L, L)] = zeros16
        idx2_v[pl.ds(i * 2 * L, L)] = zeros16
        idx2_v[pl.ds(i * 2 * L + L, L)] = zeros16
        return carry

    lax.fori_loop(0, KEPT // L, _zero, 0)

    iota16 = lax.iota(jnp.int32, L)

    def _scan(c, carry):
        m = mask_v[pl.ds(c * L, L)]
        s = plsc.cumsum(m)
        pos = carry + s - m
        vals = c * L + iota16
        keep = m != 0
        plsc.store_scatter(idx_v, [pos], vals, mask=keep)
        plsc.store_scatter(idx2_v, [2 * pos], 2 * vals, mask=keep)
        plsc.store_scatter(idx2_v, [2 * pos + 1], 2 * vals + 1, mask=keep)
        return carry + jnp.sum(m)

    lax.fori_loop(0, N // L, _scan, jnp.int32(0))

    pltpu.sync_copy(idx_v.at[pl.ds(obase, OUT_PER_W)],
                    idx_out.at[pl.ds(obase, OUT_PER_W)])
    pltpu.sync_copy(idx2_v.at[pl.ds(2 * obase, 2 * OUT_PER_W)],
                    idx2_out.at[pl.ds(2 * obase, 2 * OUT_PER_W)])

    for v in range(OUT_PER_W // L):
        ids = idx_v[pl.ds(obase + v * L, L)]
        tid_ov[pl.ds(v * L, L)] = plsc.load_gather(tid_v, [ids])
        rs_ov[pl.ds(v * L, L)] = plsc.load_gather(rs_v, [ids])
    pltpu.sync_copy(tid_ov, tid_out.at[pl.ds(obase, OUT_PER_W)])
    pltpu.sync_copy(rs_ov, rs_out.at[pl.ds(obase, OUT_PER_W)])


@functools.partial(
    pl.kernel,
    out_type=jax.ShapeDtypeStruct((KEPT, SP), jnp.float32),
    mesh=_mesh,
    scratch_types=[
        pltpu.VMEM((OUT_PER_W,), jnp.int32),
        pltpu.VMEM((2, CHUNK, SP), jnp.float32),
        pltpu.SemaphoreType.DMA,
    ],
    compiler_params=_sc_params,
)
def _sc_spec(idx_hbm, spec_hbm, spec_out, idx_v, bufs, sem):
    wid = lax.axis_index("s") * NC + lax.axis_index("c")
    obase = wid * OUT_PER_W
    pltpu.sync_copy(idx_hbm.at[pl.ds(obase, OUT_PER_W)], idx_v)

    def _gather(c, slot):
        return pltpu.async_copy(
            spec_hbm.at[idx_v.at[pl.ds(c * CHUNK, CHUNK)]],
            bufs.at[slot], sem)

    desc = [None, None]
    desc[0] = _gather(0, 0)
    for c in range(CHUNKS):
        slot = c & 1
        desc[slot].wait()
        if c + 1 < CHUNKS:
            desc[1 - slot] = _gather(c + 1, 1 - slot)
        pltpu.sync_copy(bufs.at[slot],
                        spec_out.at[pl.ds(obase + c * CHUNK, CHUNK)])


@functools.partial(
    pl.kernel,
    out_type=jax.ShapeDtypeStruct((2 * KEPT, IMGH), jnp.float32),
    mesh=_mesh,
    scratch_types=[
        pltpu.VMEM((2 * OUT_PER_W,), jnp.int32),
        pltpu.VMEM((2, CHUNK, IMGH), jnp.float32),
        pltpu.SemaphoreType.DMA,
    ],
    compiler_params=_sc_params,
)
def _sc_img(idx2_hbm, img_hbm, img_out, idx_v, bufs, sem):
    wid = lax.axis_index("s") * NC + lax.axis_index("c")
    obase = wid * 2 * OUT_PER_W
    pltpu.sync_copy(idx2_hbm.at[pl.ds(obase, 2 * OUT_PER_W)], idx_v)

    def _gather(c, slot):
        return pltpu.async_copy(
            img_hbm.at[idx_v.at[pl.ds(c * CHUNK, CHUNK)]],
            bufs.at[slot], sem)

    desc = [None, None]
    desc[0] = _gather(0, 0)
    for c in range(HCHUNKS):
        slot = c & 1
        desc[slot].wait()
        if c + 1 < HCHUNKS:
            desc[1 - slot] = _gather(c + 1, 1 - slot)
        pltpu.sync_copy(bufs.at[slot],
                        img_out.at[pl.ds(obase + c * CHUNK, CHUNK)])


def _unpad_body(pad_ref, out_ref):
    out_ref[...] = pad_ref[:, :S]


def _unpad(spec_pad_out):
    nb = KEPT // _MASK_BR
    return pl.pallas_call(
        _unpad_body,
        grid=(nb,),
        in_specs=[pl.BlockSpec((_MASK_BR, SP), lambda i: (i, 0))],
        out_specs=pl.BlockSpec((_MASK_BR, S), lambda i: (i, 0)),
        out_shape=jax.ShapeDtypeStruct((KEPT, S), jnp.float32),
    )(spec_pad_out)


def kernel(spectrum, image, targetid, redshift):
    mask, spec_pad = _compute_mask_pad(spectrum)
    imgh = image.reshape(2 * N, IMGH)
    idx, idx2, tid_o, rs_o = _sc_scan(mask, targetid, redshift)
    spec_pad_out = _sc_spec(idx, spec_pad)
    img_o = _sc_img(idx2, imgh)
    spec_o = _unpad(spec_pad_out)
    return spec_o, img_o.reshape(KEPT, 3, 64, 64), tid_o, rs_o


# speedup vs baseline: 8.3077x; 1.2063x over previous
import functools

import jax
import jax.numpy as jnp
from jax import lax
from jax.experimental import pallas as pl
from jax.experimental.pallas import tpu as pltpu
from jax.experimental.pallas import tpu_sc as plsc

N = 4096
S = 7781
KEPT = N - N // 8
NC, NS, L = 2, 16, 16
NW = NC * NS
OUT_PER_W = KEPT // NW
NVECS = OUT_PER_W // L
SCHUNK = 8
ICHUNK = 2

_MASK_BR = 128


def _mask_body(spec_ref, mask_ref):
    x = spec_ref[...]
    nz = jnp.any(x != 0.0, axis=1)
    mask_ref[0, 0, :] = nz.astype(jnp.int32)


def _compute_mask(spectrum):
    nb = N // _MASK_BR
    mask3 = pl.pallas_call(
        _mask_body,
        grid=(nb,),
        in_specs=[pl.BlockSpec((_MASK_BR, S), lambda i: (i, 0))],
        out_specs=pl.BlockSpec((1, 1, _MASK_BR), lambda i: (i, 0, 0)),
        out_shape=jax.ShapeDtypeStruct((nb, 1, _MASK_BR), jnp.int32),
    )(spectrum)
    return mask3.reshape(N)


_mesh = plsc.VectorSubcoreMesh(core_axis_name="c", subcore_axis_name="s")
_sc_params = pltpu.CompilerParams(needs_layout_passes=False)


@functools.partial(
    pl.kernel,
    out_type=(
        jax.ShapeDtypeStruct((KEPT,), jnp.int32),
        jax.ShapeDtypeStruct((KEPT,), jnp.int32),
        jax.ShapeDtypeStruct((KEPT,), jnp.float32),
    ),
    mesh=_mesh,
    scratch_types=[
        pltpu.VMEM((N,), jnp.int32),
        pltpu.VMEM((KEPT,), jnp.int32),
        pltpu.VMEM((N,), jnp.int32),
        pltpu.VMEM((N,), jnp.float32),
        pltpu.VMEM((OUT_PER_W,), jnp.int32),
        pltpu.VMEM((OUT_PER_W,), jnp.float32),
    ],
    compiler_params=_sc_params,
)
def _sc_scan(mask_hbm, tid_hbm, rs_hbm,
             idx_out, tid_out, rs_out,
             mask_v, idx_v, tid_v, rs_v, tid_ov, rs_ov):
    wid = lax.axis_index("s") * NC + lax.axis_index("c")
    obase = wid * OUT_PER_W

    pltpu.sync_copy(mask_hbm, mask_v)
    pltpu.sync_copy(tid_hbm, tid_v)
    pltpu.sync_copy(rs_hbm, rs_v)

    zeros16 = jnp.zeros((L,), jnp.int32)

    def _zero(i, carry):
        idx_v[pl.ds(i * L, L)] = zeros16
        return carry

    lax.fori_loop(0, KEPT // L, _zero, 0)

    iota16 = lax.iota(jnp.int32, L)

    def _scan(c, carry):
        m = mask_v[pl.ds(c * L, L)]
        s = plsc.cumsum(m)
        pos = carry + s - m
        vals = c * L + iota16
        plsc.store_scatter(idx_v, [pos], vals, mask=m != 0)
        return carry + jnp.sum(m)

    lax.fori_loop(0, N // L, _scan, jnp.int32(0))

    pltpu.sync_copy(idx_v.at[pl.ds(obase, OUT_PER_W)],
                    idx_out.at[pl.ds(obase, OUT_PER_W)])

    for v in range(NVECS):
        ids = idx_v[pl.ds(obase + v * L, L)]
        tid_ov[pl.ds(v * L, L)] = plsc.load_gather(tid_v, [ids])
        rs_ov[pl.ds(v * L, L)] = plsc.load_gather(rs_v, [ids])
    pltpu.sync_copy(tid_ov, tid_out.at[pl.ds(obase, OUT_PER_W)])
    pltpu.sync_copy(rs_ov, rs_out.at[pl.ds(obase, OUT_PER_W)])


@functools.partial(
    pl.kernel,
    out_type=jax.ShapeDtypeStruct((KEPT, S), jnp.float32),
    mesh=_mesh,
    scratch_types=[
        pltpu.VMEM((OUT_PER_W,), jnp.int32),
        pltpu.VMEM((2, SCHUNK, S), jnp.float32),
        pltpu.SemaphoreType.DMA,
    ],
    compiler_params=_sc_params,
)
def _sc_spec(idx_hbm, spec_hbm, spec_out, idx_v, bufs, sem):
    wid = lax.axis_index("s") * NC + lax.axis_index("c")
    obase = wid * OUT_PER_W
    pltpu.sync_copy(idx_hbm.at[pl.ds(obase, OUT_PER_W)], idx_v)
    vecs = [idx_v[pl.ds(v * L, L)] for v in range(NVECS)]

    nchunks = OUT_PER_W // SCHUNK

    def _reads(c, slot):
        vec = vecs[c // 2]
        for k in range(SCHUNK):
            r = vec[(c % 2) * SCHUNK + k]
            pltpu.async_copy(spec_hbm.at[pl.ds(r, 1)],
                             bufs.at[slot].at[pl.ds(k, 1)], sem)

    _reads(0, 0)
    for c in range(nchunks):
        slot = c & 1
        for _ in range(SCHUNK):
            pltpu.make_async_copy(spec_hbm.at[pl.ds(0, 1)],
                                  bufs.at[0].at[pl.ds(0, 1)], sem).wait()
        if c + 1 < nchunks:
            _reads(c + 1, 1 - slot)
        pltpu.sync_copy(bufs.at[slot],
                        spec_out.at[pl.ds(obase + c * SCHUNK, SCHUNK)])


@functools.partial(
    pl.kernel,
    out_type=jax.ShapeDtypeStruct((KEPT, 3, 64, 64), jnp.float32),
    mesh=_mesh,
    scratch_types=[
        pltpu.VMEM((OUT_PER_W,), jnp.int32),
        pltpu.VMEM((2, ICHUNK, 3, 64, 64), jnp.float32),
        pltpu.SemaphoreType.DMA,
    ],
    compiler_params=_sc_params,
)
def _sc_img(idx_hbm, img_hbm, img_out, idx_v, bufs, sem):
    wid = lax.axis_index("s") * NC + lax.axis_index("c")
    obase = wid * OUT_PER_W
    pltpu.sync_copy(idx_hbm.at[pl.ds(obase, OUT_PER_W)], idx_v)
    vecs = [idx_v[pl.ds(v * L, L)] for v in range(NVECS)]

    nchunks = OUT_PER_W // ICHUNK

    def _reads(c, slot):
        vec = vecs[c // 8]
        for k in range(ICHUNK):
            r = vec[(c % 8) * ICHUNK + k]
            pltpu.async_copy(img_hbm.at[pl.ds(r, 1)],
                             bufs.at[slot].at[pl.ds(k, 1)], sem)

    _reads(0, 0)
    for c in range(nchunks):
        slot = c & 1
        for _ in range(ICHUNK):
            pltpu.make_async_copy(img_hbm.at[pl.ds(0, 1)],
                                  bufs.at[0].at[pl.ds(0, 1)], sem).wait()
        if c + 1 < nchunks:
            _reads(c + 1, 1 - slot)
        pltpu.sync_copy(bufs.at[slot],
                        img_out.at[pl.ds(obase + c * ICHUNK, ICHUNK)])


def kernel(spectrum, image, targetid, redshift):
    mask = _compute_mask(spectrum)
    idx, tid_o, rs_o = _sc_scan(mask, targetid, redshift)
    spec_o = _sc_spec(idx, spectrum)
    img_o = _sc_img(idx, image)
    return spec_o, img_o, tid_o, rs_o


# speedup vs baseline: 10.6311x; 1.2797x over previous
import functools

import jax
import jax.numpy as jnp
from jax import lax
from jax.experimental import pallas as pl
from jax.experimental.pallas import tpu as pltpu
from jax.experimental.pallas import tpu_sc as plsc

N = 4096
S = 7781
IMG = 3 * 64 * 64
KEPT = N - N // 8
NC, NS, L = 2, 16, 16
NW = NC * NS
OUT_PER_W = KEPT // NW
NVECS = KEPT // L
BCHUNK = 8

_MASK_BC = 512


def _mask_body(spec_ref, mask_ref):
    x = spec_ref[...]
    nz = jnp.any(x != 0.0, axis=0)
    mask_ref[0, 0, :] = nz.astype(jnp.int32)


def _compute_mask(specT):
    nb = N // _MASK_BC
    mask3 = pl.pallas_call(
        _mask_body,
        grid=(nb,),
        in_specs=[pl.BlockSpec((S, _MASK_BC), lambda i: (0, i))],
        out_specs=pl.BlockSpec((1, 1, _MASK_BC), lambda i: (i, 0, 0)),
        out_shape=jax.ShapeDtypeStruct((nb, 1, _MASK_BC), jnp.int32),
    )(specT)
    return mask3.reshape(N)


_mesh = plsc.VectorSubcoreMesh(core_axis_name="c", subcore_axis_name="s")
_sc_params = pltpu.CompilerParams(needs_layout_passes=False)


@functools.partial(
    pl.kernel,
    out_type=(
        jax.ShapeDtypeStruct((KEPT,), jnp.int32),
        jax.ShapeDtypeStruct((KEPT,), jnp.int32),
        jax.ShapeDtypeStruct((KEPT,), jnp.float32),
    ),
    mesh=_mesh,
    scratch_types=[
        pltpu.VMEM((N,), jnp.int32),
        pltpu.VMEM((KEPT,), jnp.int32),
        pltpu.VMEM((N,), jnp.int32),
        pltpu.VMEM((N,), jnp.float32),
        pltpu.VMEM((OUT_PER_W,), jnp.int32),
        pltpu.VMEM((OUT_PER_W,), jnp.float32),
    ],
    compiler_params=_sc_params,
)
def _sc_scan(mask_hbm, tid_hbm, rs_hbm,
             idx_out, tid_out, rs_out,
             mask_v, idx_v, tid_v, rs_v, tid_ov, rs_ov):
    wid = lax.axis_index("s") * NC + lax.axis_index("c")
    obase = wid * OUT_PER_W

    pltpu.sync_copy(mask_hbm, mask_v)
    pltpu.sync_copy(tid_hbm, tid_v)
    pltpu.sync_copy(rs_hbm, rs_v)

    zeros16 = jnp.zeros((L,), jnp.int32)

    def _zero(i, carry):
        idx_v[pl.ds(i * L, L)] = zeros16
        return carry

    lax.fori_loop(0, KEPT // L, _zero, 0)

    iota16 = lax.iota(jnp.int32, L)

    def _scan(c, carry):
        m = mask_v[pl.ds(c * L, L)]
        s = plsc.cumsum(m)
        pos = carry + s - m
        vals = c * L + iota16
        plsc.store_scatter(idx_v, [pos], vals, mask=m != 0)
        return carry + jnp.sum(m)

    lax.fori_loop(0, N // L, _scan, jnp.int32(0))

    pltpu.sync_copy(idx_v.at[pl.ds(obase, OUT_PER_W)],
                    idx_out.at[pl.ds(obase, OUT_PER_W)])

    for v in range(OUT_PER_W // L):
        ids = idx_v[pl.ds(obase + v * L, L)]
        tid_ov[pl.ds(v * L, L)] = plsc.load_gather(tid_v, [ids])
        rs_ov[pl.ds(v * L, L)] = plsc.load_gather(rs_v, [ids])
    pltpu.sync_copy(tid_ov, tid_out.at[pl.ds(obase, OUT_PER_W)])
    pltpu.sync_copy(rs_ov, rs_out.at[pl.ds(obase, OUT_PER_W)])


def _make_gather_t(nbins, npairs, tail):
    nslabs = nbins // BCHUNK

    @functools.partial(
        pl.kernel,
        out_type=jax.ShapeDtypeStruct((nbins, KEPT), jnp.float32),
        mesh=_mesh,
        scratch_types=[
            pltpu.VMEM((KEPT,), jnp.int32),
            pltpu.VMEM((2, BCHUNK, N), jnp.float32),
            pltpu.VMEM((2, BCHUNK, KEPT), jnp.float32),
            pltpu.SemaphoreType.DMA,
        ],
        compiler_params=_sc_params,
    )
    def _gather(idx_hbm, in_hbm, out_hbm, idx_v, inb, outb, sem_r):
        wid = lax.axis_index("s") * NC + lax.axis_index("c")
        s0 = (wid * nslabs) // NW
        s1 = ((wid + 1) * nslabs) // NW

        pltpu.sync_copy(idx_hbm, idx_v)

        svecs = [jnp.full((L,), s, jnp.int32) for s in (0, 1)]
        bvecs = [jnp.full((L,), b, jnp.int32) for b in range(BCHUNK)]

        def _b0(ci):
            return pl.multiple_of(
                BCHUNK * jnp.minimum(s0 + ci, s1 - 1), BCHUNK)

        pltpu.async_copy(in_hbm.at[pl.ds(_b0(0), BCHUNK)], inb.at[0], sem_r)

        def _gather_slab(slot):
            def _vloop(v, c2):
                ids = idx_v[pl.ds(v * L, L)]
                for b in range(BCHUNK):
                    outb[slot, b, pl.ds(v * L, L)] = \
                        plsc.load_gather(inb, [svecs[slot], bvecs[b], ids])
                return c2

            lax.fori_loop(0, NVECS, _vloop, 0)

        def _pair(p, carry):
            for slot in (0, 1):
                ci = 2 * p + slot
                pltpu.make_async_copy(in_hbm.at[pl.ds(0, BCHUNK)],
                                      inb.at[0], sem_r).wait()
                pltpu.async_copy(in_hbm.at[pl.ds(_b0(ci + 1), BCHUNK)],
                                 inb.at[1 - slot], sem_r)
                _gather_slab(slot)
                pltpu.sync_copy(outb.at[slot],
                                out_hbm.at[pl.ds(_b0(ci), BCHUNK)])
            return carry

        lax.fori_loop(0, npairs, _pair, 0)
        pltpu.make_async_copy(in_hbm.at[pl.ds(0, BCHUNK)],
                              inb.at[0], sem_r).wait()

        if tail:
            @pl.when(wid == 0)
            def _tail():
                base = nslabs * BCHUNK
                pltpu.sync_copy(in_hbm.at[pl.ds(base, tail)],
                                inb.at[0, pl.ds(0, tail)])

                def _vloop(v, c2):
                    ids = idx_v[pl.ds(v * L, L)]
                    for b in range(tail):
                        outb[0, b, pl.ds(v * L, L)] = \
                            plsc.load_gather(inb, [svecs[0], bvecs[b], ids])
                    return c2

                lax.fori_loop(0, NVECS, _vloop, 0)
                pltpu.sync_copy(outb.at[0, pl.ds(0, tail)],
                                out_hbm.at[pl.ds(base, tail)])

    return _gather


_gather_spec = _make_gather_t(S, 16, S % BCHUNK)
_gather_img = _make_gather_t(IMG, 24, 0)


def kernel(spectrum, image, targetid, redshift):
    specT = spectrum.T
    imgT = image.transpose(1, 2, 3, 0).reshape(IMG, N)
    mask = _compute_mask(specT)
    idx, tid_o, rs_o = _sc_scan(mask, targetid, redshift)
    specT_o = _gather_spec(idx, specT)
    imgT_o = _gather_img(idx, imgT)
    spec_o = specT_o.T
    img_o = imgT_o.reshape(3, 64, 64, KEPT).transpose(3, 0, 1, 2)
    return spec_o, img_o, tid_o, rs_o


# speedup vs baseline: 13.0640x; 1.2288x over previous
import functools

import jax
import jax.numpy as jnp
from jax import lax
from jax.experimental import pallas as pl
from jax.experimental.pallas import tpu as pltpu
from jax.experimental.pallas import tpu_sc as plsc

N = 4096
S = 7781
IMG = 3 * 64 * 64
KEPT = N - N // 8
NC, NS, L = 2, 16, 16
NW = NC * NS
OUT_PER_W = KEPT // NW
NVECS = KEPT // L
BCHUNK = 8

_MASK_BC = 512


def _mask_body(spec_ref, mask_ref):
    x = spec_ref[...]
    nz = jnp.any(x != 0.0, axis=0)
    mask_ref[0, 0, :] = nz.astype(jnp.int32)


def _compute_mask(specT):
    nb = N // _MASK_BC
    mask3 = pl.pallas_call(
        _mask_body,
        grid=(nb,),
        in_specs=[pl.BlockSpec((S, _MASK_BC), lambda i: (0, i))],
        out_specs=pl.BlockSpec((1, 1, _MASK_BC), lambda i: (i, 0, 0)),
        out_shape=jax.ShapeDtypeStruct((nb, 1, _MASK_BC), jnp.int32),
    )(specT)
    return mask3.reshape(N)


_mesh = plsc.VectorSubcoreMesh(core_axis_name="c", subcore_axis_name="s")
_sc_params = pltpu.CompilerParams(needs_layout_passes=False)


@functools.partial(
    pl.kernel,
    out_type=(
        jax.ShapeDtypeStruct((KEPT,), jnp.int32),
        jax.ShapeDtypeStruct((KEPT,), jnp.int32),
        jax.ShapeDtypeStruct((KEPT,), jnp.float32),
    ),
    mesh=_mesh,
    scratch_types=[
        pltpu.VMEM((N,), jnp.int32),
        pltpu.VMEM((KEPT,), jnp.int32),
        pltpu.VMEM((N,), jnp.int32),
        pltpu.VMEM((N,), jnp.float32),
        pltpu.VMEM((OUT_PER_W,), jnp.int32),
        pltpu.VMEM((OUT_PER_W,), jnp.float32),
    ],
    compiler_params=_sc_params,
)
def _sc_scan(mask_hbm, tid_hbm, rs_hbm,
             idx_out, tid_out, rs_out,
             mask_v, idx_v, tid_v, rs_v, tid_ov, rs_ov):
    wid = lax.axis_index("s") * NC + lax.axis_index("c")
    obase = wid * OUT_PER_W

    pltpu.sync_copy(mask_hbm, mask_v)
    pltpu.sync_copy(tid_hbm, tid_v)
    pltpu.sync_copy(rs_hbm, rs_v)

    zeros16 = jnp.zeros((L,), jnp.int32)

    def _zero(i, carry):
        idx_v[pl.ds(i * L, L)] = zeros16
        return carry

    lax.fori_loop(0, KEPT // L, _zero, 0)

    iota16 = lax.iota(jnp.int32, L)

    def _scan(c, carry):
        m = mask_v[pl.ds(c * L, L)]
        s = plsc.cumsum(m)
        pos = carry + s - m
        vals = c * L + iota16
        plsc.store_scatter(idx_v, [pos], vals, mask=m != 0)
        return carry + jnp.sum(m)

    lax.fori_loop(0, N // L, _scan, jnp.int32(0))

    pltpu.sync_copy(idx_v.at[pl.ds(obase, OUT_PER_W)],
                    idx_out.at[pl.ds(obase, OUT_PER_W)])

    for v in range(OUT_PER_W // L):
        ids = idx_v[pl.ds(obase + v * L, L)]
        tid_ov[pl.ds(v * L, L)] = plsc.load_gather(tid_v, [ids])
        rs_ov[pl.ds(v * L, L)] = plsc.load_gather(rs_v, [ids])
    pltpu.sync_copy(tid_ov, tid_out.at[pl.ds(obase, OUT_PER_W)])
    pltpu.sync_copy(rs_ov, rs_out.at[pl.ds(obase, OUT_PER_W)])


def _make_gather_t(nbins, npairs, tail):
    nslabs = nbins // BCHUNK

    @functools.partial(
        pl.kernel,
        out_type=jax.ShapeDtypeStruct((nbins, KEPT), jnp.float32),
        mesh=_mesh,
        scratch_types=[
            pltpu.VMEM((KEPT,), jnp.int32),
            pltpu.VMEM((2, BCHUNK, N), jnp.float32),
            pltpu.VMEM((2, BCHUNK, KEPT), jnp.float32),
            pltpu.SemaphoreType.DMA,
            pltpu.SemaphoreType.DMA,
        ],
        compiler_params=_sc_params,
    )
    def _gather(idx_hbm, in_hbm, out_hbm, idx_v, inb, outb, sem_r, sem_w):
        wid = lax.axis_index("s") * NC + lax.axis_index("c")
        s0 = (wid * nslabs) // NW
        s1 = ((wid + 1) * nslabs) // NW

        pltpu.sync_copy(idx_hbm, idx_v)

        svecs = [jnp.full((L,), s, jnp.int32) for s in (0, 1)]
        bvecs = [jnp.full((L,), b, jnp.int32) for b in range(BCHUNK)]

        def _b0(ci):
            return pl.multiple_of(
                BCHUNK * jnp.minimum(s0 + ci, s1 - 1), BCHUNK)

        pltpu.async_copy(in_hbm.at[pl.ds(_b0(0), BCHUNK)], inb.at[0], sem_r)

        UNROLL = 4

        def _gather_slab(slot):
            def _vloop(v, c2):
                for u in range(UNROLL):
                    off = (v * UNROLL + u) * L
                    ids = idx_v[pl.ds(off, L)]
                    for b in range(BCHUNK):
                        outb[slot, b, pl.ds(off, L)] = \
                            plsc.load_gather(inb,
                                             [svecs[slot], bvecs[b], ids])
                return c2

            lax.fori_loop(0, NVECS // UNROLL, _vloop, 0)

        def _drain_w():
            pltpu.make_async_copy(out_hbm.at[pl.ds(0, BCHUNK)],
                                  outb.at[0], sem_w).wait()

        def _pair(p, carry):
            for slot in (0, 1):
                ci = 2 * p + slot
                pltpu.make_async_copy(in_hbm.at[pl.ds(0, BCHUNK)],
                                      inb.at[0], sem_r).wait()
                pltpu.async_copy(in_hbm.at[pl.ds(_b0(ci + 1), BCHUNK)],
                                 inb.at[1 - slot], sem_r)

                @pl.when(p >= 1)
                def _():
                    _drain_w()

                _gather_slab(slot)
                pltpu.async_copy(outb.at[slot],
                                 out_hbm.at[pl.ds(_b0(ci), BCHUNK)], sem_w)
            return carry

        lax.fori_loop(0, npairs, _pair, 0)
        pltpu.make_async_copy(in_hbm.at[pl.ds(0, BCHUNK)],
                              inb.at[0], sem_r).wait()
        _drain_w()
        _drain_w()

        if tail:
            @pl.when(wid == 0)
            def _tail():
                base = nslabs * BCHUNK
                pltpu.sync_copy(in_hbm.at[pl.ds(base, tail)],
                                inb.at[0, pl.ds(0, tail)])

                def _vloop(v, c2):
                    ids = idx_v[pl.ds(v * L, L)]
                    for b in range(tail):
                        outb[0, b, pl.ds(v * L, L)] = \
                            plsc.load_gather(inb, [svecs[0], bvecs[b], ids])
                    return c2

                lax.fori_loop(0, NVECS, _vloop, 0)
                pltpu.sync_copy(outb.at[0, pl.ds(0, tail)],
                                out_hbm.at[pl.ds(base, tail)])

    return _gather


_gather_spec = _make_gather_t(S, 16, S % BCHUNK)
_gather_img = _make_gather_t(IMG, 24, 0)


def kernel(spectrum, image, targetid, redshift):
    specT = spectrum.T
    imgT = image.transpose(1, 2, 3, 0).reshape(IMG, N)
    mask = _compute_mask(specT)
    idx, tid_o, rs_o = _sc_scan(mask, targetid, redshift)
    specT_o = _gather_spec(idx, specT)
    imgT_o = _gather_img(idx, imgT)
    spec_o = specT_o.T
    img_o = imgT_o.reshape(3, 64, 64, KEPT).transpose(3, 0, 1, 2)
    return spec_o, img_o, tid_o, rs_o


# speedup vs baseline: 13.1616x; 1.0075x over previous
import functools

import jax
import jax.numpy as jnp
from jax import lax
from jax.experimental import pallas as pl
from jax.experimental.pallas import tpu as pltpu
from jax.experimental.pallas import tpu_sc as plsc

N = 4096
S = 7781
IMG = 3 * 64 * 64
KEPT = N - N // 8
NC, NS, L = 2, 16, 16
NW = NC * NS
NVECS = KEPT // L
BCHUNK = 8
UNROLL = 8

_MASK_BC = 512


def _mask_body(spec_ref, mask_ref):
    x = spec_ref[...]
    nz = jnp.any(x != 0.0, axis=0)
    mask_ref[0, 0, :] = nz.astype(jnp.int32)


def _compute_mask(specT):
    nb = N // _MASK_BC
    mask3 = pl.pallas_call(
        _mask_body,
        grid=(nb,),
        in_specs=[pl.BlockSpec((S, _MASK_BC), lambda i: (0, i))],
        out_specs=pl.BlockSpec((1, 1, _MASK_BC), lambda i: (i, 0, 0)),
        out_shape=jax.ShapeDtypeStruct((nb, 1, _MASK_BC), jnp.int32),
    )(specT)
    return mask3.reshape(N)


_mesh = plsc.VectorSubcoreMesh(core_axis_name="c", subcore_axis_name="s")
_sc_params = pltpu.CompilerParams(needs_layout_passes=False)

_SLABS_S = S // BCHUNK
_SLABS_I = IMG // BCHUNK


@functools.partial(
    pl.kernel,
    out_type=(
        jax.ShapeDtypeStruct((S, KEPT), jnp.float32),
        jax.ShapeDtypeStruct((IMG, KEPT), jnp.float32),
        jax.ShapeDtypeStruct((KEPT,), jnp.float32),
        jax.ShapeDtypeStruct((KEPT,), jnp.float32),
    ),
    mesh=_mesh,
    scratch_types=[
        pltpu.VMEM((N,), jnp.int32),
        pltpu.VMEM((KEPT,), jnp.int32),
        pltpu.VMEM((2, BCHUNK, N), jnp.float32),
        pltpu.VMEM((2, BCHUNK, KEPT), jnp.float32),
        pltpu.SemaphoreType.DMA,
        pltpu.SemaphoreType.DMA,
    ],
    compiler_params=_sc_params,
)
def _sc_all(mask_hbm, spec_hbm, img_hbm, tidf_hbm, rs_hbm,
            spec_out, img_out, tidf_out, rs_out,
            mask_v, idx_v, inb, outb, sem_r, sem_w):
    wid = lax.axis_index("s") * NC + lax.axis_index("c")

    pltpu.sync_copy(mask_hbm, mask_v)

    zeros16 = jnp.zeros((L,), jnp.int32)

    def _zero(i, carry):
        idx_v[pl.ds(i * L, L)] = zeros16
        return carry

    lax.fori_loop(0, KEPT // L, _zero, 0)

    iota16 = lax.iota(jnp.int32, L)

    def _scan(c, carry):
        m = mask_v[pl.ds(c * L, L)]
        s = plsc.cumsum(m)
        pos = carry + s - m
        vals = c * L + iota16
        plsc.store_scatter(idx_v, [pos], vals, mask=m != 0)
        return carry + jnp.sum(m)

    lax.fori_loop(0, N // L, _scan, jnp.int32(0))

    svecs = [jnp.full((L,), s, jnp.int32) for s in (0, 1)]
    bvecs = [jnp.full((L,), b, jnp.int32) for b in range(BCHUNK)]

    def _gather_slab(slot):
        def _vloop(v, c2):
            for u in range(UNROLL):
                off = (v * UNROLL + u) * L
                ids = idx_v[pl.ds(off, L)]
                for b in range(BCHUNK):
                    outb[slot, b, pl.ds(off, L)] = \
                        plsc.load_gather(inb, [svecs[slot], bvecs[b], ids])
            return c2

        lax.fori_loop(0, NVECS // UNROLL, _vloop, 0)

    def _run(in_hbm, out_hbm, nslabs, npairs, tail):
        s0 = (wid * nslabs) // NW
        s1 = ((wid + 1) * nslabs) // NW

        def _b0(ci):
            return pl.multiple_of(
                BCHUNK * jnp.minimum(s0 + ci, s1 - 1), BCHUNK)

        def _drain_w():
            pltpu.make_async_copy(out_hbm.at[pl.ds(0, BCHUNK)],
                                  outb.at[0], sem_w).wait()

        pltpu.async_copy(in_hbm.at[pl.ds(_b0(0), BCHUNK)], inb.at[0],
                         sem_r)

        def _pair(p, carry):
            for slot in (0, 1):
                ci = 2 * p + slot
                pltpu.make_async_copy(in_hbm.at[pl.ds(0, BCHUNK)],
                                      inb.at[0], sem_r).wait()
                pltpu.async_copy(in_hbm.at[pl.ds(_b0(ci + 1), BCHUNK)],
                                 inb.at[1 - slot], sem_r)

                @pl.when(p >= 1)
                def _():
                    _drain_w()

                _gather_slab(slot)
                pltpu.async_copy(outb.at[slot],
                                 out_hbm.at[pl.ds(_b0(ci), BCHUNK)], sem_w)
            return carry

        lax.fori_loop(0, npairs, _pair, 0)
        pltpu.make_async_copy(in_hbm.at[pl.ds(0, BCHUNK)],
                              inb.at[0], sem_r).wait()
        _drain_w()
        _drain_w()

        if tail:
            @pl.when(wid == 0)
            def _tail():
                base = nslabs * BCHUNK
                pltpu.sync_copy(in_hbm.at[pl.ds(base, tail)],
                                inb.at[0, pl.ds(0, tail)])

                def _vloop(v, c2):
                    ids = idx_v[pl.ds(v * L, L)]
                    for b in range(tail):
                        outb[0, b, pl.ds(v * L, L)] = \
                            plsc.load_gather(inb,
                                             [svecs[0], bvecs[b], ids])
                    return c2

                lax.fori_loop(0, NVECS, _vloop, 0)
                pltpu.sync_copy(outb.at[0, pl.ds(0, tail)],
                                out_hbm.at[pl.ds(base, tail)])

    _run(spec_hbm, spec_out, _SLABS_S, 16, S % BCHUNK)
    _run(img_hbm, img_out, _SLABS_I, 24, 0)

    @pl.when(wid == 0)
    def _scalars():
        pltpu.sync_copy(tidf_hbm, inb.at[0, 0])
        pltpu.sync_copy(rs_hbm, inb.at[0, 1])

        def _vloop(v, c2):
            ids = idx_v[pl.ds(v * L, L)]
            outb[0, 0, pl.ds(v * L, L)] = \
                plsc.load_gather(inb, [svecs[0], bvecs[0], ids])
            outb[0, 1, pl.ds(v * L, L)] = \
                plsc.load_gather(inb, [svecs[0], bvecs[1], ids])
            return c2

        lax.fori_loop(0, NVECS, _vloop, 0)
        pltpu.sync_copy(outb.at[0, 0], tidf_out)
        pltpu.sync_copy(outb.at[0, 1], rs_out)


def kernel(spectrum, image, targetid, redshift):
    specT = spectrum.T
    imgT = image.transpose(1, 2, 3, 0).reshape(IMG, N)
    tidf = jax.lax.bitcast_convert_type(targetid, jnp.float32)
    mask = _compute_mask(specT)
    specT_o, imgT_o, tidf_o, rs_o = _sc_all(mask, specT, imgT, tidf,
                                            redshift)
    spec_o = specT_o.T
    img_o = imgT_o.reshape(3, 64, 64, KEPT).transpose(3, 0, 1, 2)
    tid_o = jax.lax.bitcast_convert_type(tidf_o, jnp.int32)
    return spec_o, img_o, tid_o, rs_o


# speedup vs baseline: 25.2051x; 1.9150x over previous
import functools

import jax
import jax.numpy as jnp
from jax import lax
from jax.experimental import pallas as pl
from jax.experimental.pallas import tpu as pltpu
from jax.experimental.pallas import tpu_sc as plsc

N = 4096
S = 7781
IMG = 3 * 64 * 64
KEPT = N - N // 8
NC, NS, L = 2, 16, 16
NW = NC * NS
NVECS = KEPT // L
BCHUNK = 8
UNROLL = 8

_MASK_BC = 512


def _mask_body(spec_ref, mask_ref):
    x = spec_ref[...]
    nz = jnp.any(x != 0.0, axis=0)
    mask_ref[0, 0, :] = nz.astype(jnp.int32)


def _compute_mask(specT):
    nb = N // _MASK_BC
    mask3 = pl.pallas_call(
        _mask_body,
        grid=(nb,),
        in_specs=[pl.BlockSpec((S, _MASK_BC), lambda i: (0, i))],
        out_specs=pl.BlockSpec((1, 1, _MASK_BC), lambda i: (i, 0, 0)),
        out_shape=jax.ShapeDtypeStruct((nb, 1, _MASK_BC), jnp.int32),
    )(specT)
    return mask3.reshape(N)


_mesh = plsc.VectorSubcoreMesh(core_axis_name="c", subcore_axis_name="s")
_sc_params = pltpu.CompilerParams(needs_layout_passes=False)

_SLABS_S = S // BCHUNK
_SLABS_I = IMG // BCHUNK


@functools.partial(
    pl.kernel,
    out_type=(
        jax.ShapeDtypeStruct((S, KEPT), jnp.float32),
        jax.ShapeDtypeStruct((IMG, KEPT), jnp.float32),
        jax.ShapeDtypeStruct((KEPT,), jnp.float32),
        jax.ShapeDtypeStruct((KEPT,), jnp.float32),
    ),
    mesh=_mesh,
    scratch_types=[
        pltpu.VMEM((N,), jnp.int32),
        pltpu.VMEM((KEPT,), jnp.int32),
        pltpu.VMEM((2, BCHUNK, N), jnp.float32),
        pltpu.VMEM((2, BCHUNK, KEPT), jnp.float32),
        pltpu.SemaphoreType.DMA,
        pltpu.SemaphoreType.DMA,
    ],
    compiler_params=_sc_params,
)
def _sc_all(mask_hbm, spec_hbm, img_hbm, tidf_hbm, rs_hbm,
            spec_out, img_out, tidf_out, rs_out,
            mask_v, idx_v, inb, outb, sem_r, sem_w):
    wid = lax.axis_index("s") * NC + lax.axis_index("c")

    pltpu.sync_copy(mask_hbm, mask_v)

    zeros16 = jnp.zeros((L,), jnp.int32)

    def _zero(i, carry):
        idx_v[pl.ds(i * L, L)] = zeros16
        return carry

    lax.fori_loop(0, KEPT // L, _zero, 0)

    iota16 = lax.iota(jnp.int32, L)

    def _scan(c, carry):
        m = mask_v[pl.ds(c * L, L)]
        s = plsc.cumsum(m)
        pos = carry + s - m
        vals = c * L + iota16
        plsc.store_scatter(idx_v, [pos], vals, mask=m != 0)
        return carry + jnp.sum(m)

    lax.fori_loop(0, N // L, _scan, jnp.int32(0))

    svecs = [jnp.full((L,), s, jnp.int32) for s in (0, 1)]
    bvecs = [jnp.full((L,), b, jnp.int32) for b in range(BCHUNK)]

    def _gather_slab(slot):
        def _vloop(v, c2):
            for u in range(UNROLL):
                off = (v * UNROLL + u) * L
                ids = idx_v[pl.ds(off, L)]
                vals = [plsc.load_gather(inb, [svecs[slot], bvecs[b], ids])
                        for b in range(BCHUNK)]
                for b in range(BCHUNK):
                    outb[slot, b, pl.ds(off, L)] = vals[b]
            return c2

        lax.fori_loop(0, NVECS // UNROLL, _vloop, 0)

    def _run(in_hbm, out_hbm, nslabs, npairs, tail):
        s0 = (wid * nslabs) // NW
        s1 = ((wid + 1) * nslabs) // NW

        def _b0(ci):
            return pl.multiple_of(
                BCHUNK * jnp.minimum(s0 + ci, s1 - 1), BCHUNK)

        def _drain_w():
            pltpu.make_async_copy(out_hbm.at[pl.ds(0, BCHUNK)],
                                  outb.at[0], sem_w).wait()

        pltpu.async_copy(in_hbm.at[pl.ds(_b0(0), BCHUNK)], inb.at[0],
                         sem_r)

        def _pair(p, carry):
            for slot in (0, 1):
                ci = 2 * p + slot
                pltpu.make_async_copy(in_hbm.at[pl.ds(0, BCHUNK)],
                                      inb.at[0], sem_r).wait()
                pltpu.async_copy(in_hbm.at[pl.ds(_b0(ci + 1), BCHUNK)],
                                 inb.at[1 - slot], sem_r)

                @pl.when(p >= 1)
                def _():
                    _drain_w()

                _gather_slab(slot)
                pltpu.async_copy(outb.at[slot],
                                 out_hbm.at[pl.ds(_b0(ci), BCHUNK)], sem_w)
            return carry

        lax.fori_loop(0, npairs, _pair, 0)
        pltpu.make_async_copy(in_hbm.at[pl.ds(0, BCHUNK)],
                              inb.at[0], sem_r).wait()
        _drain_w()
        _drain_w()

        if tail:
            @pl.when(wid == 0)
            def _tail():
                base = nslabs * BCHUNK
                pltpu.sync_copy(in_hbm.at[pl.ds(base, tail)],
                                inb.at[0, pl.ds(0, tail)])

                def _vloop(v, c2):
                    ids = idx_v[pl.ds(v * L, L)]
                    for b in range(tail):
                        outb[0, b, pl.ds(v * L, L)] = \
                            plsc.load_gather(inb,
                                             [svecs[0], bvecs[b], ids])
                    return c2

                lax.fori_loop(0, NVECS, _vloop, 0)
                pltpu.sync_copy(outb.at[0, pl.ds(0, tail)],
                                out_hbm.at[pl.ds(base, tail)])

    _run(spec_hbm, spec_out, _SLABS_S, 16, S % BCHUNK)
    _run(img_hbm, img_out, _SLABS_I, 24, 0)

    @pl.when(wid == 0)
    def _scalars():
        pltpu.sync_copy(tidf_hbm, inb.at[0, 0])
        pltpu.sync_copy(rs_hbm, inb.at[0, 1])

        def _vloop(v, c2):
            ids = idx_v[pl.ds(v * L, L)]
            outb[0, 0, pl.ds(v * L, L)] = \
                plsc.load_gather(inb, [svecs[0], bvecs[0], ids])
            outb[0, 1, pl.ds(v * L, L)] = \
                plsc.load_gather(inb, [svecs[0], bvecs[1], ids])
            return c2

        lax.fori_loop(0, NVECS, _vloop, 0)
        pltpu.sync_copy(outb.at[0, 0], tidf_out)
        pltpu.sync_copy(outb.at[0, 1], rs_out)


def kernel(spectrum, image, targetid, redshift):
    specT = spectrum.T
    imgT = image.transpose(1, 2, 3, 0).reshape(IMG, N)
    tidf = jax.lax.bitcast_convert_type(targetid, jnp.float32)
    mask = _compute_mask(specT)
    specT_o, imgT_o, tidf_o, rs_o = _sc_all(mask, specT, imgT, tidf,
                                            redshift)
    spec_o = specT_o.T
    img_o = imgT_o.reshape(3, 64, 64, KEPT).transpose(3, 0, 1, 2)
    tid_o = jax.lax.bitcast_convert_type(tidf_o, jnp.int32)
    return spec_o, img_o, tid_o, rs_o
